# Initial kernel scaffold; baseline (speedup 1.0000x reference)
import jax
import jax.numpy as jnp
from jax.experimental import pallas as pl


def kernel(node_feats, feat_sum, node_pair_feat, W1, W2, b2, W3, b3, edge_index):
    raise NotImplementedError("write your pallas kernel here")


# speedup vs baseline: 2.9938x; 2.9938x over previous
import functools

import jax
import jax.numpy as jnp
from jax import lax
from jax.experimental import pallas as pl
from jax.experimental.pallas import tpu as pltpu
from jax.experimental.pallas import tpu_sc as plsc

V, E, D, DP = 10000, 320000, 128, 16
VP = 10240

NC, NS, L = 2, 16, 16
NW = NC * NS
EPW = E // NW
C = 80
NCH = EPW // C
G = C // L

BE = 3200


def _attn_body(fs_ref, npf_ref, w1_ref, w2_ref, b2_ref, w3_ref, b3_ref, o_ref):
    h = jnp.dot(fs_ref[...], w1_ref[...], preferred_element_type=jnp.float32)
    h = h + jnp.dot(npf_ref[...], w2_ref[...], preferred_element_type=jnp.float32)
    h = h + b2_ref[...]
    h = jnp.maximum(h, 0.0)
    s = jnp.dot(h, w3_ref[...], preferred_element_type=jnp.float32)
    o_ref[...] = jax.nn.sigmoid(s + b3_ref[...])


def _attention(feat_sum, node_pair_feat, W1, W2, b2, W3, b3):
    grid = (E // BE,)
    return pl.pallas_call(
        _attn_body,
        grid=grid,
        in_specs=[
            pl.BlockSpec((BE, D), lambda i: (i, 0)),
            pl.BlockSpec((BE, DP), lambda i: (i, 0)),
            pl.BlockSpec((D, D), lambda i: (0, 0)),
            pl.BlockSpec((DP, D), lambda i: (0, 0)),
            pl.BlockSpec((1, D), lambda i: (0, 0)),
            pl.BlockSpec((D, 1), lambda i: (0, 0)),
            pl.BlockSpec((1, 1), lambda i: (0, 0)),
        ],
        out_specs=pl.BlockSpec((BE, 1), lambda i: (i, 0)),
        out_shape=jax.ShapeDtypeStruct((E, 1), jnp.float32),
    )(feat_sum, node_pair_feat, W1, W2, b2.reshape(1, D), W3, b3.reshape(1, 1))


def _sc_body(nf_hbm, a_hbm, src_hbm, dst_hbm, zero_hbm, out_hbm,
             src_v, dst_v, a_v, rows_v, ctx_sh, sem):
    c = lax.axis_index("c")
    s = lax.axis_index("s")
    wid = c * NS + s

    pltpu.sync_copy(a_hbm.at[wid], a_v)

    rows_per_sub = VP // NS
    pltpu.sync_copy(zero_hbm.at[pl.ds(s * rows_per_sub, rows_per_sub)],
                    ctx_sh.at[pl.ds(s * rows_per_sub, rows_per_sub)])
    plsc.subcore_barrier()

    def chunk_body(i, carry):
        pltpu.sync_copy(src_hbm.at[wid, i], src_v)
        pltpu.sync_copy(dst_hbm.at[wid, i], dst_v)
        pltpu.async_copy(nf_hbm.at[src_v], rows_v, sem).wait()

        def group_body(g, carry2):
            a16 = a_v[i, pl.ds(g * L, L)]
            for e in range(L):
                a_sp = lax.gather(
                    a16, jnp.full((L, 1), e, dtype=jnp.int32),
                    lax.GatherDimensionNumbers(offset_dims=(),
                                               collapsed_slice_dims=(0,),
                                               start_index_map=(0,)),
                    slice_sizes=(1,),
                    mode=lax.GatherScatterMode.PROMISE_IN_BOUNDS)
                row = g * L + e
                for j in range(D // L):
                    sl = pl.ds(j * L, L)
                    rows_v[row, sl] = rows_v[row, sl] * a_sp
            return carry2

        lax.fori_loop(0, G, group_body, 0, unroll=False)

        pltpu.sync_copy(rows_v, ctx_sh.at[dst_v], add=True)
        return carry

    lax.fori_loop(0, NCH, chunk_body, 0, unroll=False)

    plsc.subcore_barrier()
    pltpu.sync_copy(ctx_sh.at[pl.ds(s * rows_per_sub, rows_per_sub)],
                    out_hbm.at[c, pl.ds(s * rows_per_sub, rows_per_sub)])


def _sc_scatter(node_feats, a, src, dst, zero):
    mesh = plsc.VectorSubcoreMesh(core_axis_name="c", subcore_axis_name="s")
    run = pl.kernel(
        _sc_body,
        out_type=jax.ShapeDtypeStruct((NC, VP, D), jnp.float32),
        mesh=mesh,
        scratch_types=[
            pltpu.VMEM((C,), jnp.int32),
            pltpu.VMEM((C,), jnp.int32),
            pltpu.VMEM((NCH, C), jnp.float32),
            pltpu.VMEM((C, D), jnp.float32),
            pltpu.VMEM_SHARED((VP, D), jnp.float32),
            pltpu.SemaphoreType.DMA,
        ],
    )
    return run(node_feats, a, src, dst, zero)


VB = 2000


def _add_body(p_ref, o_ref):
    o_ref[...] = p_ref[0] + p_ref[1]


def _add_partials(parts):
    return pl.pallas_call(
        _add_body,
        grid=(V // VB,),
        in_specs=[pl.BlockSpec((NC, VB, D), lambda i: (0, i, 0))],
        out_specs=pl.BlockSpec((VB, D), lambda i: (i, 0)),
        out_shape=jax.ShapeDtypeStruct((V, D), jnp.float32),
    )(parts)


def kernel(node_feats, feat_sum, node_pair_feat, W1, W2, b2, W3, b3, edge_index):
    a = _attention(feat_sum, node_pair_feat, W1, W2, b2, W3, b3)
    a3 = a.reshape(NW, NCH, C)
    src3 = edge_index[0].reshape(NW, NCH, C)
    dst3 = edge_index[1].reshape(NW, NCH, C)
    zero = jnp.zeros((VP, D), jnp.float32)
    parts = _sc_scatter(node_feats, a3, src3, dst3, zero)
    return _add_partials(parts)
